# baseline (speedup 1.0000x reference)
import functools

import jax
import jax.numpy as jnp
from jax import lax
from jax.experimental import pallas as pl
from jax.experimental.pallas import tpu as pltpu
from jax.experimental.pallas import tpu_sc as plsc

_NC = 2
_NS = 16
_NW = _NC * _NS
_C = 128


_WIN = 4
_PAIRS = (7, 13)


def _segment_partials(table, src_r, dst_r, pairs0, pairs1):
    W = table.shape[1]
    n_pad = table.shape[0] - 1
    rps = n_pad // _NS
    nfull, rem = divmod(rps, _C)

    mesh = plsc.VectorSubcoreMesh(core_axis_name="c", subcore_axis_name="s")

    @functools.partial(
        pl.kernel,
        mesh=mesh,
        compiler_params=pltpu.CompilerParams(use_tc_tiling_on_sc=False),
        out_type=jax.ShapeDtypeStruct((2 * n_pad, W), jnp.float32),
        scratch_types=[
            pltpu.VMEM((_WIN, _C), jnp.int32),
            pltpu.VMEM((_WIN, _C), jnp.int32),
            pltpu.VMEM((_WIN, _C), jnp.int32),
            pltpu.VMEM((_WIN, _C), jnp.int32),
            pltpu.VMEM((_C, W), jnp.float32),
            pltpu.VMEM_SHARED((n_pad, W), jnp.float32),
            pltpu.SemaphoreType.DMA,
            pltpu.SemaphoreType.DMA,
            pltpu.SemaphoreType.DMA,
        ],
    )
    def sc_kernel(table_hbm, src_hbm, dst_hbm, out_hbm,
                  ws0, ws1, wd0, wd1, buf, acc,
                  sem, wsem0, wsem1):
        cid = lax.axis_index("c")
        sid = lax.axis_index("s")
        wid = sid * _NC + cid
        wsrc = (ws0, ws1)
        wdst = (wd0, wd1)
        wsems = (wsem0, wsem1)
        nw2 = jnp.where(cid == 0, pairs0, pairs1)

        zvec = jnp.zeros((16,), jnp.float32)

        def _zrow(i, carry):
            for j in range(W // 16):
                buf[i, pl.ds(j * 16, 16)] = zvec
            return carry

        lax.fori_loop(0, _C, _zrow, 0)

        def _zcopy(t, carry):
            pltpu.sync_copy(buf, acc.at[pl.ds(sid * rps + t * _C, _C)])
            return carry

        lax.fori_loop(0, nfull, _zcopy, 0)
        if rem:
            pltpu.sync_copy(buf.at[pl.ds(0, rem)],
                            acc.at[pl.ds(sid * rps + nfull * _C, rem)])
        plsc.subcore_barrier()

        def _load_win(w, wp):
            pltpu.async_copy(src_hbm.at[wid, pl.ds(w * _WIN, _WIN)],
                             wsrc[wp], wsems[wp])
            pltpu.async_copy(dst_hbm.at[wid, pl.ds(w * _WIN, _WIN)],
                             wdst[wp], wsems[wp])

        def _wait_win(w, wp):
            pltpu.make_async_copy(src_hbm.at[wid, pl.ds(w * _WIN, _WIN)],
                                  wsrc[wp], wsems[wp]).wait()
            pltpu.make_async_copy(dst_hbm.at[wid, pl.ds(w * _WIN, _WIN)],
                                  wdst[wp], wsems[wp]).wait()

        _load_win(0, 0)
        _load_win(1, 1)

        def _pair(w2, carry):
            for wp in range(2):
                w = w2 * 2 + wp
                _wait_win(w, wp)
                ws, wd = wsrc[wp], wdst[wp]
                for k in range(_WIN):
                    pltpu.async_copy(table_hbm.at[ws.at[k]], buf, sem).wait()
                    pltpu.sync_copy(buf, acc.at[wd.at[k]], add=True)
                _load_win(w + 2, wp)
            return carry

        lax.fori_loop(0, nw2, _pair, 0)
        _wait_win(2 * nw2, 0)
        _wait_win(2 * nw2 + 1, 1)

        plsc.subcore_barrier()
        base = sid * rps
        pltpu.sync_copy(acc.at[pl.ds(base, rps)],
                        out_hbm.at[pl.ds(cid * n_pad + base, rps)])

    return sc_kernel(table, src_r, dst_r)


_BLK = 1000


def _mm(a, b):
    return jnp.dot(a, b, precision=jax.lax.Precision.HIGHEST)


def _layer_mid(x, pa, pb, da, db, W_self0, W_neigh0, b0, W_neigh1):
    n, d = x.shape
    grid = (n // _BLK,)

    def body(x_ref, pa_ref, pb_ref, da_ref, db_ref,
             ws_ref, wn_ref, b_ref, wn1_ref, h_ref, p1_ref):
        deg = jnp.sum(da_ref[...] + db_ref[...], axis=1, keepdims=True) * (1.0 / 16.0)
        inv = 1.0 / jnp.maximum(deg, 1.0)
        agg = (pa_ref[...] + pb_ref[...]) * inv
        h = jnp.maximum(_mm(x_ref[...], ws_ref[...]) + _mm(agg, wn_ref[...])
                        + b_ref[...], 0.0)
        h_ref[...] = h
        p1_ref[...] = _mm(h, wn1_ref[...])

    row = pl.BlockSpec((_BLK, d), lambda i: (i, 0))
    row16 = pl.BlockSpec((_BLK, 16), lambda i: (i, 0))
    full = pl.BlockSpec((d, d), lambda i: (0, 0))
    vec = pl.BlockSpec((1, d), lambda i: (0, 0))
    return pl.pallas_call(
        body,
        grid=grid,
        in_specs=[row, row, row, row16, row16, full, full, vec, full],
        out_specs=[row, row],
        out_shape=[jax.ShapeDtypeStruct((n, d), jnp.float32),
                   jax.ShapeDtypeStruct((n, d), jnp.float32)],
    )(x, pa, pb, da, db, W_self0, W_neigh0, b0[None, :], W_neigh1)


def _layer_out(h, qa, qb, da, db, W_self1, b1):
    n, d = h.shape
    grid = (n // _BLK,)

    def body(h_ref, qa_ref, qb_ref, da_ref, db_ref, ws_ref, b_ref, o_ref):
        deg = jnp.sum(da_ref[...] + db_ref[...], axis=1, keepdims=True) * (1.0 / 16.0)
        inv = 1.0 / jnp.maximum(deg, 1.0)
        o_ref[...] = (_mm(h_ref[...], ws_ref[...])
                      + (qa_ref[...] + qb_ref[...]) * inv + b_ref[...])

    row = pl.BlockSpec((_BLK, d), lambda i: (i, 0))
    row16 = pl.BlockSpec((_BLK, 16), lambda i: (i, 0))
    full = pl.BlockSpec((d, d), lambda i: (0, 0))
    vec = pl.BlockSpec((1, d), lambda i: (0, 0))
    return pl.pallas_call(
        body,
        grid=grid,
        in_specs=[row, row, row, row16, row16, full, vec],
        out_specs=row,
        out_shape=jax.ShapeDtypeStruct((n, d), jnp.float32),
    )(h, qa, qb, da, db, W_self1, b1[None, :])


def kernel(features, edge_index, W_self0, W_neigh0, b0, W_self1, W_neigh1, b1):
    n, d = features.shape
    e = edge_index.shape[1]
    src = edge_index[0]
    dst = edge_index[1]

    assert n % _NS == 0

    p0, p1c = _PAIRS
    epw = 2 * _WIN * _C
    cap0, cap1 = p0 * epw * _NS, p1c * epw * _NS
    assert cap0 + cap1 >= e
    e0 = min(cap0, e)
    e1 = e - e0
    n_chunks_max = (max(p0, p1c) + 1) * 2 * _WIN

    def _blocks(vals, lo, hi, cap, pairs, fill_spread):
        part = vals[lo:hi]
        padn = cap - (hi - lo)
        if fill_spread:
            fill = jnp.arange(padn, dtype=jnp.int32) % n
        else:
            fill = jnp.full((padn,), n, jnp.int32)
        return jnp.concatenate([part, fill]).reshape(_NS, pairs * 2 * _WIN, _C)

    src_full = jnp.zeros((_NW, n_chunks_max, _C), jnp.int32)
    dst_full = jnp.zeros((_NW, n_chunks_max, _C), jnp.int32)
    src_full = src_full.at[0::2, :p0 * 2 * _WIN].set(
        _blocks(src, 0, e0, cap0, p0, False))
    dst_full = dst_full.at[0::2, :p0 * 2 * _WIN].set(
        _blocks(dst, 0, e0, cap0, p0, True))
    src_full = src_full.at[1::2, :p1c * 2 * _WIN].set(
        _blocks(src, e0, e, cap1, p1c, False))
    dst_full = dst_full.at[1::2, :p1c * 2 * _WIN].set(
        _blocks(dst, e0, e, cap1, p1c, True))

    aug = jnp.concatenate(
        [jnp.concatenate([features, jnp.ones((n, 16), jnp.float32)], axis=1),
         jnp.zeros((1, d + 16), jnp.float32)], axis=0)
    part0 = _segment_partials(aug, src_full, dst_full, p0, p1c)
    pa, da = part0[:n, :d], part0[:n, d:]
    pb, db = part0[n:, :d], part0[n:, d:]

    h, p1 = _layer_mid(features, pa, pb, da, db, W_self0, W_neigh0, b0, W_neigh1)

    p1t = jnp.concatenate([p1, jnp.zeros((1, d), jnp.float32)], axis=0)
    part1 = _segment_partials(p1t, src_full, dst_full, p0, p1c)
    qa = part1[:n]
    qb = part1[n:]

    return _layer_out(h, qa, qb, da, db, W_self1, b1)


# speedup vs baseline: 2.1710x; 2.1710x over previous
import functools

import jax
import jax.numpy as jnp
from jax import lax
from jax.experimental import pallas as pl
from jax.experimental.pallas import tpu as pltpu
from jax.experimental.pallas import tpu_sc as plsc

_NC = 2
_NS = 16
_NW = _NC * _NS
_C = 128


def _segment_partials(table, src_r, dst_r):
    n_chunks = src_r.shape[1]
    W = table.shape[1]
    n_pad = table.shape[0] - 1
    rps = n_pad // _NS
    nfull, rem = divmod(rps, _C)

    mesh = plsc.VectorSubcoreMesh(core_axis_name="c", subcore_axis_name="s")

    @functools.partial(
        pl.kernel,
        mesh=mesh,
        compiler_params=pltpu.CompilerParams(use_tc_tiling_on_sc=False),
        out_type=jax.ShapeDtypeStruct((2 * n_pad, W), jnp.float32),
        scratch_types=[
            pltpu.VMEM((n_chunks, _C), jnp.int32),
            pltpu.VMEM((n_chunks, _C), jnp.int32),
            pltpu.VMEM((_C, W), jnp.float32),
            pltpu.VMEM_SHARED((n_pad, W), jnp.float32),
            pltpu.SemaphoreType.DMA,
        ],
    )
    def sc_kernel(table_hbm, src_hbm, dst_hbm, out_hbm,
                  src_v, dst_v, buf, acc, sem):
        cid = lax.axis_index("c")
        sid = lax.axis_index("s")
        wid = sid * _NC + cid

        zvec = jnp.zeros((16,), jnp.float32)

        def _zrow(i, carry):
            for j in range(W // 16):
                buf[i, pl.ds(j * 16, 16)] = zvec
            return carry

        lax.fori_loop(0, _C, _zrow, 0)

        def _zcopy(t, carry):
            pltpu.sync_copy(buf, acc.at[pl.ds(sid * rps + t * _C, _C)])
            return carry

        lax.fori_loop(0, nfull, _zcopy, 0)
        if rem:
            pltpu.sync_copy(buf.at[pl.ds(0, rem)],
                            acc.at[pl.ds(sid * rps + nfull * _C, rem)])
        plsc.subcore_barrier()

        pltpu.sync_copy(src_hbm.at[wid], src_v)
        pltpu.sync_copy(dst_hbm.at[wid], dst_v)

        def _edge_chunk(j, carry):
            pltpu.async_copy(table_hbm.at[src_v.at[j]], buf, sem).wait()
            pltpu.sync_copy(buf, acc.at[dst_v.at[j]], add=True)
            return carry

        lax.fori_loop(0, n_chunks, _edge_chunk, 0)

        plsc.subcore_barrier()
        base = sid * rps
        pltpu.sync_copy(acc.at[pl.ds(base, rps)],
                        out_hbm.at[pl.ds(cid * n_pad + base, rps)])

    return sc_kernel(table, src_r, dst_r)


_BLK = 1000


def _mm(a, b):
    return jnp.dot(a, b, precision=jax.lax.Precision.HIGHEST)


def _layer_mid(x, pa, pb, da, db, W_self0, W_neigh0, b0, W_neigh1):
    n, d = x.shape
    grid = (n // _BLK,)

    def body(x_ref, pa_ref, pb_ref, da_ref, db_ref,
             ws_ref, wn_ref, b_ref, wn1_ref, h_ref, p1_ref):
        deg = jnp.sum(da_ref[...] + db_ref[...], axis=1, keepdims=True) * (1.0 / 16.0)
        inv = 1.0 / jnp.maximum(deg, 1.0)
        agg = (pa_ref[...] + pb_ref[...]) * inv
        h = jnp.maximum(_mm(x_ref[...], ws_ref[...]) + _mm(agg, wn_ref[...])
                        + b_ref[...], 0.0)
        h_ref[...] = h
        p1_ref[...] = _mm(h, wn1_ref[...])

    row = pl.BlockSpec((_BLK, d), lambda i: (i, 0))
    row16 = pl.BlockSpec((_BLK, 16), lambda i: (i, 0))
    full = pl.BlockSpec((d, d), lambda i: (0, 0))
    vec = pl.BlockSpec((1, d), lambda i: (0, 0))
    return pl.pallas_call(
        body,
        grid=grid,
        in_specs=[row, row, row, row16, row16, full, full, vec, full],
        out_specs=[row, row],
        out_shape=[jax.ShapeDtypeStruct((n, d), jnp.float32),
                   jax.ShapeDtypeStruct((n, d), jnp.float32)],
    )(x, pa, pb, da, db, W_self0, W_neigh0, b0[None, :], W_neigh1)


def _layer_out(h, qa, qb, da, db, W_self1, b1):
    n, d = h.shape
    grid = (n // _BLK,)

    def body(h_ref, qa_ref, qb_ref, da_ref, db_ref, ws_ref, b_ref, o_ref):
        deg = jnp.sum(da_ref[...] + db_ref[...], axis=1, keepdims=True) * (1.0 / 16.0)
        inv = 1.0 / jnp.maximum(deg, 1.0)
        o_ref[...] = (_mm(h_ref[...], ws_ref[...])
                      + (qa_ref[...] + qb_ref[...]) * inv + b_ref[...])

    row = pl.BlockSpec((_BLK, d), lambda i: (i, 0))
    row16 = pl.BlockSpec((_BLK, 16), lambda i: (i, 0))
    full = pl.BlockSpec((d, d), lambda i: (0, 0))
    vec = pl.BlockSpec((1, d), lambda i: (0, 0))
    return pl.pallas_call(
        body,
        grid=grid,
        in_specs=[row, row, row, row16, row16, full, vec],
        out_specs=row,
        out_shape=jax.ShapeDtypeStruct((n, d), jnp.float32),
    )(h, qa, qb, da, db, W_self1, b1[None, :])


def kernel(features, edge_index, W_self0, W_neigh0, b0, W_self1, W_neigh1, b1):
    n, d = features.shape
    e = edge_index.shape[1]
    src = edge_index[0]
    dst = edge_index[1]

    assert n % _NS == 0
    n_chunks = -(-e // (_NW * _C))
    e_pad = _NW * n_chunks * _C

    pad_dst = jnp.arange(e_pad - e, dtype=jnp.int32) % n
    src_r = jnp.concatenate(
        [src, jnp.full((e_pad - e,), n, jnp.int32)]).reshape(_NW, n_chunks, _C)
    dst_r = jnp.concatenate([dst, pad_dst]).reshape(_NW, n_chunks, _C)

    aug = jnp.concatenate(
        [jnp.concatenate([features, jnp.ones((n, 16), jnp.float32)], axis=1),
         jnp.zeros((1, d + 16), jnp.float32)], axis=0)
    part0 = _segment_partials(aug, src_r, dst_r)
    pa, da = part0[:n, :d], part0[:n, d:]
    pb, db = part0[n:, :d], part0[n:, d:]

    h, p1 = _layer_mid(features, pa, pb, da, db, W_self0, W_neigh0, b0, W_neigh1)

    p1t = jnp.concatenate([p1, jnp.zeros((1, d), jnp.float32)], axis=0)
    part1 = _segment_partials(p1t, src_r, dst_r)
    qa = part1[:n]
    qb = part1[n:]

    return _layer_out(h, qa, qb, da, db, W_self1, b1)


# speedup vs baseline: 2.2042x; 1.0153x over previous
import functools

import jax
import jax.numpy as jnp
from jax import lax
from jax.experimental import pallas as pl
from jax.experimental.pallas import tpu as pltpu
from jax.experimental.pallas import tpu_sc as plsc

_NC = 2
_NS = 16
_NW = _NC * _NS
_C = 128


def _segment_partials(table, src_r, dst_r, with_deg):
    n_chunks = src_r.shape[1]
    W = table.shape[1]
    n_pad = table.shape[0] + _NS
    rps = n_pad // _NS
    nfull, rem = divmod(rps, _C)

    mesh = plsc.VectorSubcoreMesh(core_axis_name="c", subcore_axis_name="s")

    out_types = [jax.ShapeDtypeStruct((2 * n_pad, W), jnp.float32)]
    scratch = [
        pltpu.VMEM((n_chunks, _C), jnp.int32),
        pltpu.VMEM((n_chunks, _C), jnp.int32),
        pltpu.VMEM((_C, W), jnp.float32),
        pltpu.VMEM_SHARED((n_pad, W), jnp.float32),
        pltpu.SemaphoreType.DMA,
    ]
    if with_deg:
        out_types.append(jax.ShapeDtypeStruct((2 * n_pad, 16), jnp.float32))
        scratch += [
            pltpu.VMEM((_C, 16), jnp.float32),
            pltpu.VMEM_SHARED((n_pad, 16), jnp.float32),
        ]

    @functools.partial(
        pl.kernel,
        mesh=mesh,
        compiler_params=pltpu.CompilerParams(use_tc_tiling_on_sc=False),
        out_type=out_types,
        scratch_types=scratch,
    )
    def sc_kernel(table_hbm, src_hbm, dst_hbm, *rest):
        if with_deg:
            out_hbm, deg_hbm, src_v, dst_v, buf, acc, sem, ones, dacc = rest
        else:
            out_hbm, src_v, dst_v, buf, acc, sem = rest
        cid = lax.axis_index("c")
        sid = lax.axis_index("s")
        wid = sid * _NC + cid
        base = sid * rps

        zvec = jnp.zeros((16,), jnp.float32)

        def _zrow(i, carry):
            for j in range(W // 16):
                buf[i, pl.ds(j * 16, 16)] = zvec
            if with_deg:
                ones[i, pl.ds(0, 16)] = zvec
            return carry

        lax.fori_loop(0, _C, _zrow, 0)

        def _zcopy(t, carry):
            pltpu.sync_copy(buf, acc.at[pl.ds(base + t * _C, _C)])
            if with_deg:
                pltpu.sync_copy(ones, dacc.at[pl.ds(base + t * _C, _C)])
            return carry

        lax.fori_loop(0, nfull, _zcopy, 0)
        if rem:
            pltpu.sync_copy(buf.at[pl.ds(0, rem)],
                            acc.at[pl.ds(base + nfull * _C, rem)])
            if with_deg:
                pltpu.sync_copy(ones.at[pl.ds(0, rem)],
                                dacc.at[pl.ds(base + nfull * _C, rem)])

        if with_deg:
            ovec = jnp.ones((16,), jnp.float32)

            def _orow(i, carry):
                ones[i, pl.ds(0, 16)] = ovec
                return carry

            lax.fori_loop(0, _C, _orow, 0)

        pltpu.sync_copy(src_hbm.at[wid], src_v)
        pltpu.sync_copy(dst_hbm.at[wid], dst_v)
        plsc.subcore_barrier()

        def _edge_chunk(j, carry):
            pltpu.async_copy(table_hbm.at[src_v.at[j]], buf, sem).wait()
            pltpu.sync_copy(buf, acc.at[dst_v.at[j]], add=True)
            if with_deg:
                pltpu.sync_copy(ones, dacc.at[dst_v.at[j]], add=True)
            return carry

        lax.fori_loop(0, n_chunks, _edge_chunk, 0)

        plsc.subcore_barrier()
        pltpu.sync_copy(acc.at[pl.ds(base, rps)],
                        out_hbm.at[pl.ds(cid * n_pad + base, rps)])
        if with_deg:
            pltpu.sync_copy(dacc.at[pl.ds(base, rps)],
                            deg_hbm.at[pl.ds(cid * n_pad + base, rps)])

    res = sc_kernel(table, src_r, dst_r)
    return res if with_deg else res[0]


_BLK = 1000


def _mm(a, b):
    return jnp.dot(a, b, precision=jax.lax.Precision.HIGHEST)


def _layer_mid(x, pa, pb, da, db, W_self0, W_neigh0, b0, W_neigh1):
    n, d = x.shape
    grid = (n // _BLK,)

    def body(x_ref, pa_ref, pb_ref, da_ref, db_ref,
             ws_ref, wn_ref, b_ref, wn1_ref, h_ref, p1_ref):
        deg = jnp.sum(da_ref[...] + db_ref[...], axis=1, keepdims=True) * (1.0 / 16.0)
        inv = 1.0 / jnp.maximum(deg, 1.0)
        agg = (pa_ref[...] + pb_ref[...]) * inv
        h = jnp.maximum(_mm(x_ref[...], ws_ref[...]) + _mm(agg, wn_ref[...])
                        + b_ref[...], 0.0)
        h_ref[...] = h
        p1_ref[...] = _mm(h, wn1_ref[...])

    row = pl.BlockSpec((_BLK, d), lambda i: (i, 0))
    row16 = pl.BlockSpec((_BLK, 16), lambda i: (i, 0))
    full = pl.BlockSpec((d, d), lambda i: (0, 0))
    vec = pl.BlockSpec((1, d), lambda i: (0, 0))
    return pl.pallas_call(
        body,
        grid=grid,
        in_specs=[row, row, row, row16, row16, full, full, vec, full],
        out_specs=[row, row],
        out_shape=[jax.ShapeDtypeStruct((n, d), jnp.float32),
                   jax.ShapeDtypeStruct((n, d), jnp.float32)],
    )(x, pa, pb, da, db, W_self0, W_neigh0, b0[None, :], W_neigh1)


def _layer_out(h, qa, qb, da, db, W_self1, b1):
    n, d = h.shape
    grid = (n // _BLK,)

    def body(h_ref, qa_ref, qb_ref, da_ref, db_ref, ws_ref, b_ref, o_ref):
        deg = jnp.sum(da_ref[...] + db_ref[...], axis=1, keepdims=True) * (1.0 / 16.0)
        inv = 1.0 / jnp.maximum(deg, 1.0)
        o_ref[...] = (_mm(h_ref[...], ws_ref[...])
                      + (qa_ref[...] + qb_ref[...]) * inv + b_ref[...])

    row = pl.BlockSpec((_BLK, d), lambda i: (i, 0))
    row16 = pl.BlockSpec((_BLK, 16), lambda i: (i, 0))
    full = pl.BlockSpec((d, d), lambda i: (0, 0))
    vec = pl.BlockSpec((1, d), lambda i: (0, 0))
    return pl.pallas_call(
        body,
        grid=grid,
        in_specs=[row, row, row, row16, row16, full, vec],
        out_specs=row,
        out_shape=jax.ShapeDtypeStruct((n, d), jnp.float32),
    )(h, qa, qb, da, db, W_self1, b1[None, :])


def kernel(features, edge_index, W_self0, W_neigh0, b0, W_self1, W_neigh1, b1):
    n, d = features.shape
    e = edge_index.shape[1]
    src = edge_index[0]
    dst = edge_index[1]

    assert n % _NS == 0
    n_pad = n + _NS
    n_chunks = -(-e // (_NW * _C))
    e_pad = _NW * n_chunks * _C

    pad_dst = n + (jnp.arange(e_pad - e, dtype=jnp.int32) % _NS)
    src_r = jnp.concatenate(
        [src, jnp.zeros((e_pad - e,), jnp.int32)]).reshape(_NW, n_chunks, _C)
    dst_r = jnp.concatenate([dst, pad_dst]).reshape(_NW, n_chunks, _C)

    part0, deg = _segment_partials(features, src_r, dst_r, True)
    pa, da = part0[:n], deg[:n]
    pb, db = part0[n_pad:n_pad + n], deg[n_pad:n_pad + n]

    h, p1 = _layer_mid(features, pa, pb, da, db, W_self0, W_neigh0, b0, W_neigh1)

    part1 = _segment_partials(p1, src_r, dst_r, False)
    qa = part1[:n]
    qb = part1[n_pad:n_pad + n]

    return _layer_out(h, qa, qb, da, db, W_self1, b1)
